# baseline (speedup 1.0000x reference)
import jax
import jax.numpy as jnp
from jax.experimental import pallas as pl

D_MODEL = 768
D_FF = 4 * D_MODEL
N_TOKENS = 8192
TOKEN_BLOCK = 512


def _fuse_kernel(w2_ref, w3_ref, v_ref):
    v_ref[...] = jax.lax.dot_general(
        w3_ref[...], w2_ref[...],
        dimension_numbers=(((2,), (1,)), ((0,), (0,))),
        preferred_element_type=jnp.float32,
    )


def _moe_kernel(x_ref, gate_t_ref, w1t_ref, b1_ref, vt_ref, w3t_ref,
                b2_ref, w3_ref, b3_ref, out_ref):
    x = x_ref[...]
    logits = jnp.dot(x, gate_t_ref[...],
                     preferred_element_type=jnp.float32)
    m1 = jnp.max(logits, axis=1, keepdims=True)
    iota = jax.lax.broadcasted_iota(jnp.int32, logits.shape, 1)
    first_max = jnp.min(jnp.where(logits == m1, iota, logits.shape[1]),
                        axis=1, keepdims=True)
    masked = jnp.where(iota == first_max, -jnp.inf, logits)
    m2 = jnp.max(masked, axis=1, keepdims=True)
    rw0 = 1.0 / (1.0 + jnp.exp(m2 - m1))
    rw1 = 1.0 - rw0

    c = jnp.sum(b2_ref[...] * w3_ref[...], axis=1, keepdims=True) \
        + b3_ref[...]

    xw3 = jnp.dot(x, w3t_ref[...],
                  preferred_element_type=jnp.float32)

    s = []
    for j in range(2):
        h = jnp.dot(x, w1t_ref[j],
                    preferred_element_type=jnp.float32)
        h = jnp.maximum(h + b1_ref[j][None, :], 0.0)
        sj = jnp.dot(h.astype(vt_ref.dtype), vt_ref[...],
                     preferred_element_type=jnp.float32)
        s.append(sj[:, j:j + 1] + xw3[:, j:j + 1] + c[j:j + 1, 0:1])

    out_ref[...] = rw0 * s[0] + rw1 * s[1]


def kernel(hidden_states, gate_w, W1, b1, W2, b2, W3, b3):
    n, d = hidden_states.shape
    f = D_FF

    v = pl.pallas_call(
        _fuse_kernel,
        out_shape=jax.ShapeDtypeStruct((2, 1, f), jnp.float32),
    )(W2[:2], W3[:2])

    bf = jnp.bfloat16
    vt = v.reshape(2, f).T.astype(bf)
    w1t = W1[:2].transpose(0, 2, 1).astype(bf)
    w3t = W3[:2].reshape(2, d).T.astype(bf)
    x_bf = hidden_states.astype(bf)
    gate_t = gate_w.T.astype(bf)

    nb = n // TOKEN_BLOCK
    out = pl.pallas_call(
        _moe_kernel,
        grid=(nb,),
        in_specs=[
            pl.BlockSpec((TOKEN_BLOCK, d), lambda i: (i, 0)),
            pl.BlockSpec((d, gate_w.shape[0]), lambda i: (0, 0)),
            pl.BlockSpec((2, d, f), lambda i: (0, 0, 0)),
            pl.BlockSpec((2, f), lambda i: (0, 0)),
            pl.BlockSpec((f, 2), lambda i: (0, 0)),
            pl.BlockSpec((d, 2), lambda i: (0, 0)),
            pl.BlockSpec((2, d), lambda i: (0, 0)),
            pl.BlockSpec((2, d), lambda i: (0, 0)),
            pl.BlockSpec((2, 1), lambda i: (0, 0)),
        ],
        out_specs=pl.BlockSpec((TOKEN_BLOCK, 1), lambda i: (i, 0)),
        out_shape=jax.ShapeDtypeStruct((n, 1), jnp.float32),
    )(x_bf, gate_t, w1t, b1[:2], vt, w3t,
      b2[:2], W3[:2].reshape(2, d), b3[:2])
    return out


# speedup vs baseline: 1.6866x; 1.6866x over previous
import jax
import jax.numpy as jnp
from jax.experimental import pallas as pl
from jax.experimental.pallas import tpu as pltpu

D_MODEL = 768
D_FF = 4 * D_MODEL
TOKEN_BLOCK = 512

_NT = (((1,), (1,)), ((), ()))


def _fuse_kernel(w2_ref, w3_ref, v_ref):
    v_ref[...] = jax.lax.dot_general(
        w3_ref[...], w2_ref[...],
        dimension_numbers=(((2,), (1,)), ((0,), (0,))),
        preferred_element_type=jnp.float32,
    )


def _moe_kernel(x_ref, gate_ref, w1_ref, b1_ref, v_ref, w3_ref,
                b2_ref, b3_ref, out_ref, w1bf_ref):
    i = pl.program_id(0)

    @pl.when(i == 0)
    def _cast_weights():
        w1bf_ref[...] = w1_ref[...].astype(jnp.bfloat16)

    x = x_ref[...]
    xb = x.astype(jnp.bfloat16)

    logits = jax.lax.dot_general(x, gate_ref[...], _NT,
                                 preferred_element_type=jnp.float32)
    m1 = jnp.max(logits, axis=1, keepdims=True)
    iota = jax.lax.broadcasted_iota(jnp.int32, logits.shape, 1)
    first_max = jnp.min(jnp.where(logits == m1, iota, logits.shape[1]),
                        axis=1, keepdims=True)
    masked = jnp.where(iota == first_max, -jnp.inf, logits)
    m2 = jnp.max(masked, axis=1, keepdims=True)
    rw0 = 1.0 / (1.0 + jnp.exp(m2 - m1))
    rw1 = 1.0 - rw0

    w3m = w3_ref[:, 0, :]
    c = jnp.sum(b2_ref[:, 0, :] * w3m, axis=1, keepdims=True) \
        + b3_ref[:, 0, :]

    xw3 = jax.lax.dot_general(x, w3m, _NT,
                              preferred_element_type=jnp.float32)

    s = []
    for j in range(2):
        h = jax.lax.dot_general(xb, w1bf_ref[j], _NT,
                                preferred_element_type=jnp.float32)
        h = jnp.maximum(h + b1_ref[j], 0.0)
        sj = jnp.sum(h * v_ref[j], axis=1, keepdims=True)
        s.append(sj + xw3[:, j:j + 1] + c[j:j + 1, 0:1])

    out_ref[...] = rw0 * s[0] + rw1 * s[1]


def kernel(hidden_states, gate_w, W1, b1, W2, b2, W3, b3):
    n, d = hidden_states.shape
    f = D_FF
    e = gate_w.shape[0]

    v = pl.pallas_call(
        _fuse_kernel,
        grid=(1,),
        in_specs=[
            pl.BlockSpec((2, d, f), lambda i: (0, 0, 0)),
            pl.BlockSpec((2, 1, d), lambda i: (0, 0, 0)),
        ],
        out_specs=pl.BlockSpec((2, 1, f), lambda i: (0, 0, 0)),
        out_shape=jax.ShapeDtypeStruct((2, 1, f), jnp.float32),
    )(W2, W3)

    nb = n // TOKEN_BLOCK
    out = pl.pallas_call(
        _moe_kernel,
        grid=(nb,),
        in_specs=[
            pl.BlockSpec((TOKEN_BLOCK, d), lambda i: (i, 0)),
            pl.BlockSpec((e, d), lambda i: (0, 0)),
            pl.BlockSpec((2, f, d), lambda i: (0, 0, 0)),
            pl.BlockSpec((2, 1, f), lambda i: (0, 0, 0)),
            pl.BlockSpec((2, 1, f), lambda i: (0, 0, 0)),
            pl.BlockSpec((2, 1, d), lambda i: (0, 0, 0)),
            pl.BlockSpec((2, 1, d), lambda i: (0, 0, 0)),
            pl.BlockSpec((2, 1, 1), lambda i: (0, 0, 0)),
        ],
        out_specs=pl.BlockSpec((TOKEN_BLOCK, 1), lambda i: (i, 0)),
        out_shape=jax.ShapeDtypeStruct((n, 1), jnp.float32),
        scratch_shapes=[pltpu.VMEM((2, f, d), jnp.bfloat16)],
    )(hidden_states, gate_w, W1, b1.reshape(e, 1, f), v,
      W3, b2.reshape(e, 1, d), b3.reshape(e, 1, 1))
    return out


# speedup vs baseline: 1.6915x; 1.0029x over previous
import jax
import jax.numpy as jnp
from jax.experimental import pallas as pl
from jax.experimental.pallas import tpu as pltpu

D_MODEL = 768
D_FF = 4 * D_MODEL
TOKEN_BLOCK = 512

_NT = (((1,), (1,)), ((), ()))


def _fuse_kernel(w2_ref, w3_ref, v_ref):
    v_ref[...] = jax.lax.dot_general(
        w3_ref[...], w2_ref[...],
        dimension_numbers=(((2,), (1,)), ((0,), (0,))),
        preferred_element_type=jnp.float32,
    )


def _moe_kernel(x_ref, gate_ref, w1_ref, b1_ref, v_ref, w3_ref,
                b2_ref, b3_ref, out_ref):
    x = x_ref[...]
    xb = x.astype(jnp.bfloat16)

    logits = jax.lax.dot_general(x, gate_ref[...], _NT,
                                 preferred_element_type=jnp.float32)
    m1 = jnp.max(logits, axis=1, keepdims=True)
    iota = jax.lax.broadcasted_iota(jnp.int32, logits.shape, 1)
    first_max = jnp.min(jnp.where(logits == m1, iota, logits.shape[1]),
                        axis=1, keepdims=True)
    masked = jnp.where(iota == first_max, -jnp.inf, logits)
    m2 = jnp.max(masked, axis=1, keepdims=True)
    rw0 = 1.0 / (1.0 + jnp.exp(m2 - m1))
    rw1 = 1.0 - rw0

    w3m = w3_ref[:, 0, :]
    c = jnp.sum(b2_ref[:, 0, :] * w3m, axis=1, keepdims=True) \
        + b3_ref[:, 0, :]

    xw3 = jax.lax.dot_general(x, w3m, _NT,
                              preferred_element_type=jnp.float32)

    s = []
    for j in range(2):
        h = jax.lax.dot_general(x, w1_ref[j], _NT,
                                preferred_element_type=jnp.float32)
        h = jnp.maximum(h + b1_ref[j], 0.0)
        sj = jnp.sum(h * v_ref[j], axis=1, keepdims=True)
        s.append(sj + xw3[:, j:j + 1] + c[j:j + 1, 0:1])

    out_ref[...] = rw0 * s[0] + rw1 * s[1]


def kernel(hidden_states, gate_w, W1, b1, W2, b2, W3, b3):
    n, d = hidden_states.shape
    f = D_FF
    e = gate_w.shape[0]

    v = pl.pallas_call(
        _fuse_kernel,
        grid=(1,),
        in_specs=[
            pl.BlockSpec((2, d, f), lambda i: (0, 0, 0)),
            pl.BlockSpec((2, 1, d), lambda i: (0, 0, 0)),
        ],
        out_specs=pl.BlockSpec((2, 1, f), lambda i: (0, 0, 0)),
        out_shape=jax.ShapeDtypeStruct((2, 1, f), jnp.float32),
    )(W2, W3)

    nb = n // TOKEN_BLOCK
    out = pl.pallas_call(
        _moe_kernel,
        grid=(nb,),
        in_specs=[
            pl.BlockSpec((TOKEN_BLOCK, d), lambda i: (i, 0)),
            pl.BlockSpec((e, d), lambda i: (0, 0)),
            pl.BlockSpec((2, f, d), lambda i: (0, 0, 0)),
            pl.BlockSpec((2, 1, f), lambda i: (0, 0, 0)),
            pl.BlockSpec((2, 1, f), lambda i: (0, 0, 0)),
            pl.BlockSpec((2, 1, d), lambda i: (0, 0, 0)),
            pl.BlockSpec((2, 1, d), lambda i: (0, 0, 0)),
            pl.BlockSpec((2, 1, 1), lambda i: (0, 0, 0)),
        ],
        out_specs=pl.BlockSpec((TOKEN_BLOCK, 1), lambda i: (i, 0)),
        out_shape=jax.ShapeDtypeStruct((n, 1), jnp.float32),
    )(hidden_states, gate_w, W1, b1.reshape(e, 1, f), v,
      W3, b2.reshape(e, 1, d), b3.reshape(e, 1, 1))
    return out
